# baseline (speedup 1.0000x reference)
import functools

import jax
import jax.numpy as jnp
from jax import lax
from jax.experimental import pallas as pl
from jax.experimental.pallas import tpu as pltpu
from jax.experimental.pallas import tpu_sc as plsc

_L = 16
_SC_TOKENS = 1024
_TC_BT = 64
_PRE_BT = 512


def _softmax_matvec_body(logits_ref, cen_ref, probs_ref, ta_ref):
    l = logits_ref[...]
    m = jnp.max(l, axis=-1, keepdims=True)
    e = jnp.exp(l - m)
    s = jnp.sum(e, axis=-1, keepdims=True)
    p = e / s
    probs_ref[...] = p
    ta_ref[...] = jnp.dot(p, cen_ref[...], preferred_element_type=jnp.float32)


@functools.lru_cache(maxsize=None)
def _make_tc_softmax(sc_tokens: int, p_dim: int, nc: int, ndv: int, bt: int):
    grid = sc_tokens // bt
    return pl.pallas_call(
        _softmax_matvec_body,
        grid=(grid,),
        in_specs=[
            pl.BlockSpec((bt, nc), lambda i: (i, 0)),
            pl.BlockSpec((nc, ndv), lambda i: (0, 0)),
        ],
        out_specs=[
            pl.BlockSpec((bt, nc), lambda i: (i, 0)),
            pl.BlockSpec((bt, ndv), lambda i: (i, 0)),
        ],
        out_shape=[
            jax.ShapeDtypeStruct((sc_tokens, nc), jnp.float32),
            jax.ShapeDtypeStruct((sc_tokens, ndv), jnp.float32),
        ],
    )


def _tc_full_body(row_ref, cen_ref, out_ref):
    bt = row_ref.shape[0]
    nc = cen_ref.shape[0]
    ndv = cen_ref.shape[1]
    per_lane = 128 // ndv
    row = row_ref[...]
    logits = row[:, :nc]
    m = jnp.max(logits, axis=-1, keepdims=True)
    e = jnp.exp(logits - m)
    p = e / jnp.sum(e, axis=-1, keepdims=True)
    nmacro = nc // 128
    width = 128 * 128 // per_lane
    kk = lax.broadcasted_iota(jnp.int32, (128, width), 0)
    cc = lax.broadcasted_iota(jnp.int32, (128, width), 1)
    sel = (kk == (cc // 128) * per_lane + (cc % 128) // ndv).astype(jnp.float32)
    acc = jnp.zeros((bt, 128), jnp.float32)
    for q in range(nmacro):
        pq = p[:, 128 * q : 128 * (q + 1)]
        pb = jnp.dot(pq, sel, preferred_element_type=jnp.float32)
        muq = row[:, nc + width * q : nc + width * (q + 1)]
        acc = acc + jnp.sum((pb * muq).reshape(bt, width // 128, 128), axis=1)
    red = acc[:, :ndv]
    for j in range(1, per_lane):
        red = red + acc[:, ndv * j : ndv * (j + 1)]
    out_ref[...] = red + jnp.dot(p, cen_ref[...], preferred_element_type=jnp.float32)


@functools.lru_cache(maxsize=None)
def _make_tc_full(tokens: int, p_dim: int, nc: int, ndv: int, start: int, bt: int):
    grid = tokens // bt
    blk0 = start // bt
    return pl.pallas_call(
        _tc_full_body,
        grid=(grid,),
        in_specs=[
            pl.BlockSpec((bt, p_dim), lambda i: (i + blk0, 0)),
            pl.BlockSpec((nc, ndv), lambda i: (0, 0)),
        ],
        out_specs=pl.BlockSpec((bt, ndv), lambda i: (i, 0)),
        out_shape=jax.ShapeDtypeStruct((tokens, ndv), jnp.float32),
    )


@functools.lru_cache(maxsize=None)
def _make_sc_weighted_sum(sc_tokens: int, p_dim: int, nc: int, ndv: int):
    info = plsc.get_sparse_core_info()
    ncores, nsub = info.num_cores, info.num_subcores
    nw = ncores * nsub
    tpw = sc_tokens // nw
    assert tpw % 2 == 0
    mu_off = nc
    mu_len = nc * ndv
    nchunk = nc // _L
    mesh = plsc.VectorSubcoreMesh(core_axis_name="c", subcore_axis_name="s")

    @functools.partial(
        pl.kernel,
        out_type=jax.ShapeDtypeStruct((sc_tokens * ndv,), jnp.float32),
        mesh=mesh,
        scratch_types=[
            pltpu.VMEM((4, mu_len), jnp.float32),
            pltpu.VMEM((tpw, nc), jnp.float32),
            pltpu.VMEM((tpw * ndv,), jnp.float32),
            pltpu.SemaphoreType.DMA,
            pltpu.SemaphoreType.DMA,
            pltpu.SemaphoreType.DMA,
            pltpu.SemaphoreType.DMA,
            pltpu.SemaphoreType.DMA,
        ],
    )
    def sc_kernel(
        params_hbm, probs_hbm, out_hbm, mu_v, pr_v, out_v, sem0, sem1, sem2, sem3, semp
    ):
        wid = lax.axis_index("s") * ncores + lax.axis_index("c")
        base = wid * tpw
        sems = (sem0, sem1, sem2, sem3)

        pr_cp = pltpu.make_async_copy(probs_hbm.at[pl.ds(base, tpw)], pr_v, semp)
        pr_cp.start()

        def mu_cp(tok, b):
            return pltpu.make_async_copy(
                params_hbm.at[tok, pl.ds(mu_off, mu_len)], mu_v.at[b], sems[b]
            )

        def compute(tok_local, b):
            mu = mu_v.at[b]

            def kc_body(kc, accs):
                a0, a1, b0, b1 = accs
                e = pr_v[tok_local, pl.ds(kc * _L, _L)]
                for j in range(_L):
                    pb = e[j]
                    row = (kc * _L + j) * ndv
                    if j % 2 == 0:
                        a0 = a0 + pb * mu[pl.ds(row, _L)]
                        a1 = a1 + pb * mu[pl.ds(row + _L, _L)]
                    else:
                        b0 = b0 + pb * mu[pl.ds(row, _L)]
                        b1 = b1 + pb * mu[pl.ds(row + _L, _L)]
                return a0, a1, b0, b1

            zeros = jnp.zeros((_L,), jnp.float32)
            a0, a1, b0, b1 = lax.fori_loop(
                0, nchunk, kc_body, (zeros, zeros, zeros, zeros)
            )
            out_v[pl.ds(tok_local * ndv, _L)] = a0 + b0
            out_v[pl.ds(tok_local * ndv + _L, _L)] = a1 + b1

        nbuf = 4
        for b in range(nbuf):
            mu_cp(base + b, b).start()
        pr_cp.wait()

        def outer(i, carry):
            t = base + nbuf * i
            for b in range(nbuf):
                mu_cp(t + b, b).wait()
                compute(nbuf * i + b, b)
                mu_cp(t + b + nbuf, b).start()
            return carry

        lax.fori_loop(0, tpw // nbuf - 1, outer, 0)
        for b in range(nbuf):
            mu_cp(base + tpw - nbuf + b, b).wait()
            compute(tpw - nbuf + b, b)
        pltpu.sync_copy(out_v, out_hbm.at[pl.ds(base * ndv, tpw * ndv)])

    return sc_kernel


def kernel(params, centres):
    B, T, p_dim = params.shape
    nc, ndv = centres.shape
    tokens = B * T
    s = _SC_TOKENS
    p2 = params.reshape(tokens, p_dim)
    probs, term_a = _make_tc_softmax(s, p_dim, nc, ndv, _PRE_BT)(p2, centres)
    term_b = _make_sc_weighted_sum(s, p_dim, nc, ndv)(p2, probs)
    out_sc = term_a + term_b.reshape(s, ndv)
    out_tc = _make_tc_full(tokens - s, p_dim, nc, ndv, s, _TC_BT)(p2, centres)
    out = jnp.concatenate([out_sc, out_tc], axis=0)
    return out.reshape(B, T, ndv)


# speedup vs baseline: 1.0328x; 1.0328x over previous
import functools

import jax
import jax.numpy as jnp
from jax import lax
from jax.experimental import pallas as pl
from jax.experimental.pallas import tpu as pltpu
from jax.experimental.pallas import tpu_sc as plsc

_L = 16
_SC_TOKENS = 1024
_TC_BT = 64
_PRE_BT = 512


def _softmax_matvec_body(logits_ref, cent_ref, probs_ref, ta_ref):
    l = logits_ref[...]
    m = jnp.max(l, axis=-1, keepdims=True)
    e = jnp.exp(l - m)
    s = jnp.sum(e, axis=-1, keepdims=True)
    p = e / s
    probs_ref[...] = p
    ta_ref[...] = lax.dot_general(
        p, cent_ref[...], (((1,), (1,)), ((), ())),
        preferred_element_type=jnp.float32,
    )


@functools.lru_cache(maxsize=None)
def _make_tc_softmax(sc_tokens: int, p_dim: int, nc: int, ndv: int, bt: int):
    grid = sc_tokens // bt
    return pl.pallas_call(
        _softmax_matvec_body,
        grid=(grid,),
        in_specs=[
            pl.BlockSpec((bt, nc), lambda i: (i, 0)),
            pl.BlockSpec((ndv, nc), lambda i: (0, 0)),
        ],
        out_specs=[
            pl.BlockSpec((bt, nc), lambda i: (i, 0)),
            pl.BlockSpec((bt, ndv), lambda i: (i, 0)),
        ],
        out_shape=[
            jax.ShapeDtypeStruct((sc_tokens, nc), jnp.float32),
            jax.ShapeDtypeStruct((sc_tokens, ndv), jnp.float32),
        ],
    )


def _tc_full_body(row_ref, cent_ref, out_ref):
    bt = row_ref.shape[0]
    ndv = cent_ref.shape[0]
    nc = cent_ref.shape[1]
    per_lane = 128 // ndv
    row = row_ref[...]
    logits = row[:, :nc]
    m = jnp.max(logits, axis=-1, keepdims=True)
    e = jnp.exp(logits - m)
    p = e / jnp.sum(e, axis=-1, keepdims=True)
    nmacro = nc // 128
    width = 128 * 128 // per_lane
    kk = lax.broadcasted_iota(jnp.int32, (128, width), 0)
    cc = lax.broadcasted_iota(jnp.int32, (128, width), 1)
    sel = (kk == (cc // 128) * per_lane + (cc % 128) // ndv).astype(jnp.float32)
    acc = jnp.zeros((bt, 128), jnp.float32)
    for q in range(nmacro):
        pq = p[:, 128 * q : 128 * (q + 1)]
        pb = jnp.dot(pq, sel, preferred_element_type=jnp.float32)
        muq = row[:, nc + width * q : nc + width * (q + 1)]
        acc = acc + jnp.sum((pb * muq).reshape(bt, width // 128, 128), axis=1)
    red = acc[:, :ndv]
    for j in range(1, per_lane):
        red = red + acc[:, ndv * j : ndv * (j + 1)]
    out_ref[...] = red + lax.dot_general(
        p, cent_ref[...], (((1,), (1,)), ((), ())),
        preferred_element_type=jnp.float32,
    )


@functools.lru_cache(maxsize=None)
def _make_tc_full(tokens: int, p_dim: int, nc: int, ndv: int, start: int, bt: int):
    grid = tokens // bt
    blk0 = start // bt
    return pl.pallas_call(
        _tc_full_body,
        grid=(grid,),
        in_specs=[
            pl.BlockSpec((bt, p_dim), lambda i: (i + blk0, 0)),
            pl.BlockSpec((ndv, nc), lambda i: (0, 0)),
        ],
        out_specs=pl.BlockSpec((bt, ndv), lambda i: (i, 0)),
        out_shape=jax.ShapeDtypeStruct((tokens, ndv), jnp.float32),
    )


@functools.lru_cache(maxsize=None)
def _make_sc_weighted_sum(sc_tokens: int, p_dim: int, nc: int, ndv: int):
    info = plsc.get_sparse_core_info()
    ncores, nsub = info.num_cores, info.num_subcores
    nw = ncores * nsub
    tpw = sc_tokens // nw
    assert tpw % 2 == 0
    mu_off = nc
    mu_len = nc * ndv
    nchunk = nc // _L
    mesh = plsc.VectorSubcoreMesh(core_axis_name="c", subcore_axis_name="s")

    @functools.partial(
        pl.kernel,
        out_type=jax.ShapeDtypeStruct((sc_tokens * ndv,), jnp.float32),
        mesh=mesh,
        scratch_types=[
            pltpu.VMEM((2, mu_len), jnp.float32),
            pltpu.VMEM((tpw, nc), jnp.float32),
            pltpu.VMEM((tpw * ndv,), jnp.float32),
            pltpu.SemaphoreType.DMA,
            pltpu.SemaphoreType.DMA,
            pltpu.SemaphoreType.DMA,
        ],
    )
    def sc_kernel(params_hbm, probs_hbm, out_hbm, mu_v, pr_v, out_v, sem0, sem1, semp):
        wid = lax.axis_index("s") * ncores + lax.axis_index("c")
        base = wid * tpw
        sems = (sem0, sem1)

        pr_cp = pltpu.make_async_copy(probs_hbm.at[pl.ds(base, tpw)], pr_v, semp)
        pr_cp.start()

        def mu_cp(tok, b):
            return pltpu.make_async_copy(
                params_hbm.at[tok, pl.ds(mu_off, mu_len)], mu_v.at[b], sems[b]
            )

        def compute(tok_local, b):
            mu = mu_v.at[b]

            def kc_body(kc, accs):
                a0, a1, b0, b1 = accs
                e = pr_v[tok_local, pl.ds(kc * _L, _L)]
                for j in range(_L):
                    pb = e[j]
                    row = (kc * _L + j) * ndv
                    if j % 2 == 0:
                        a0 = a0 + pb * mu[pl.ds(row, _L)]
                        a1 = a1 + pb * mu[pl.ds(row + _L, _L)]
                    else:
                        b0 = b0 + pb * mu[pl.ds(row, _L)]
                        b1 = b1 + pb * mu[pl.ds(row + _L, _L)]
                return a0, a1, b0, b1

            zeros = jnp.zeros((_L,), jnp.float32)
            a0, a1, b0, b1 = lax.fori_loop(
                0, nchunk, kc_body, (zeros, zeros, zeros, zeros)
            )
            out_v[pl.ds(tok_local * ndv, _L)] = a0 + b0
            out_v[pl.ds(tok_local * ndv + _L, _L)] = a1 + b1

        nbuf = 2
        for b in range(nbuf):
            mu_cp(base + b, b).start()
        pr_cp.wait()

        def outer(i, carry):
            t = base + nbuf * i
            for b in range(nbuf):
                mu_cp(t + b, b).wait()
                compute(nbuf * i + b, b)
                mu_cp(t + b + nbuf, b).start()
            return carry

        lax.fori_loop(0, tpw // nbuf - 1, outer, 0)
        for b in range(nbuf):
            mu_cp(base + tpw - nbuf + b, b).wait()
            compute(tpw - nbuf + b, b)
        pltpu.sync_copy(out_v, out_hbm.at[pl.ds(base * ndv, tpw * ndv)])

    return sc_kernel


def kernel(params, centres):
    B, T, p_dim = params.shape
    nc, ndv = centres.shape
    tokens = B * T
    s = _SC_TOKENS
    p2 = params.reshape(tokens, p_dim)
    cent = centres.T
    probs, term_a = _make_tc_softmax(s, p_dim, nc, ndv, _PRE_BT)(p2, cent)
    term_b = _make_sc_weighted_sum(s, p_dim, nc, ndv)(p2, probs)
    out_sc = term_a + term_b.reshape(s, ndv)
    out_tc = _make_tc_full(tokens - s, p_dim, nc, ndv, s, _TC_BT)(p2, cent)
    out = jnp.concatenate([out_sc, out_tc], axis=0)
    return out.reshape(B, T, ndv)
